# Initial kernel scaffold; baseline (speedup 1.0000x reference)
import jax
import jax.numpy as jnp
from jax.experimental import pallas as pl


def kernel(x, edge_index, batch, W1, b1, W2, b2, W3, b3, lw1, lb1, lw2, lb2):
    raise NotImplementedError("write your pallas kernel here")


# speedup vs baseline: 9.3309x; 9.3309x over previous
import functools

import jax
import jax.numpy as jnp
from jax import lax
from jax.experimental import pallas as pl
from jax.experimental.pallas import tpu as pltpu
from jax.experimental.pallas import tpu_sc as plsc

N = 10000
D = 128
E = 320000
NP = 10240
EP = 327680
EPR = EP // 128
NW = 32
NBLK = EPR // NW
NBUF = 2
CH = 8
NCH = NBLK // CH
TROWS = NP // 16
DEGW = 16
ROWB = 1024
GRID = NP // ROWB


@functools.lru_cache(maxsize=None)
def _sc_kernels():
    mesh = plsc.VectorSubcoreMesh(
        core_axis_name="c", subcore_axis_name="s", num_cores=2,
        num_subcores=16)

    @functools.partial(
        pl.kernel,
        out_type=jax.ShapeDtypeStruct((2, NP, DEGW), jnp.float32),
        mesh=mesh,
        scratch_types=[
            pltpu.VMEM_SHARED((NP, DEGW), jnp.float32),
            pltpu.VMEM((NBLK, 128), jnp.int32),
            pltpu.VMEM((128, DEGW), jnp.float32),
        ],
    )
    def deg_kernel(dst_hbm, out_hbm, deg_sh, idx_v, val_v):
        cid = lax.axis_index("c")
        sid = lax.axis_index("s")
        wid = sid * 2 + cid

        def _fill(v):
            def body(i, carry):
                val_v[i, :] = jnp.full((DEGW,), v, jnp.float32)
                return carry
            lax.fori_loop(0, 128, body, 0)

        _fill(0.0)
        for k in range(TROWS // 128):
            pltpu.sync_copy(val_v, deg_sh.at[pl.ds(sid * TROWS + k * 128, 128)])
        _fill(1.0)
        pltpu.sync_copy(dst_hbm.at[pl.ds(wid * NBLK, NBLK)], idx_v)
        plsc.subcore_barrier()

        def body(j, carry):
            pltpu.sync_copy(val_v, deg_sh.at[idx_v.at[j]], add=True)
            return carry
        lax.fori_loop(0, NBLK, body, 0)

        plsc.subcore_barrier()
        pltpu.sync_copy(deg_sh.at[pl.ds(sid * TROWS, TROWS)],
                        out_hbm.at[cid, pl.ds(sid * TROWS, TROWS)])

    @functools.partial(
        pl.kernel,
        out_type=jax.ShapeDtypeStruct((2, NP, D), jnp.float32),
        mesh=mesh,
        scratch_types=[
            pltpu.VMEM_SHARED((NP, D), jnp.float32),
            pltpu.VMEM((2, CH, 128), jnp.int32),
            pltpu.VMEM((2, CH, 128), jnp.int32),
            pltpu.VMEM((NBUF, 128, D), jnp.float32),
            pltpu.SemaphoreType.DMA,
            pltpu.SemaphoreType.DMA,
            pltpu.SemaphoreType.DMA,
        ],
    )
    def agg_kernel(g_hbm, src_hbm, dst_hbm, out_hbm,
                   acc_sh, sidx, didx, bufs, sg0, sg1, si):
        sems = [sg0, sg1]
        cid = lax.axis_index("c")
        sid = lax.axis_index("s")
        wid = sid * 2 + cid
        base = wid * NBLK

        def zrow(r, carry):
            def zcol(c, inner):
                bufs[0, r, pl.ds(c * 16, 16)] = jnp.zeros((16,), jnp.float32)
                return inner
            lax.fori_loop(0, 8, zcol, 0)
            return carry
        lax.fori_loop(0, 128, zrow, 0)
        for k in range(TROWS // 128):
            pltpu.sync_copy(bufs.at[0],
                            acc_sh.at[pl.ds(sid * TROWS + k * 128, 128)])

        pltpu.sync_copy(src_hbm.at[pl.ds(base, CH)], sidx.at[0])
        pltpu.sync_copy(dst_hbm.at[pl.ds(base, CH)], didx.at[0])
        plsc.subcore_barrier()

        for b in range(NBUF):
            pltpu.async_copy(g_hbm.at[sidx.at[0, b]], bufs.at[b], sems[b])

        for c in range(NCH):
            pc = c % 2
            if c < NCH - 1:
                nc = (c + 1) % 2
                pltpu.async_copy(
                    src_hbm.at[pl.ds(base + (c + 1) * CH, CH)], sidx.at[nc], si)
                pltpu.async_copy(
                    dst_hbm.at[pl.ds(base + (c + 1) * CH, CH)], didx.at[nc], si)
            for r in range(CH):
                j = c * CH + r
                b = j % NBUF
                pltpu.make_async_copy(
                    g_hbm.at[sidx.at[pc, r]], bufs.at[b], sems[b]).wait()
                pltpu.sync_copy(bufs.at[b], acc_sh.at[didx.at[pc, r]], add=True)
                nj = j + NBUF
                if nj < NBLK:
                    ncc = nj // CH
                    if ncc != c and nj % CH == 0:
                        pltpu.make_async_copy(
                            src_hbm.at[pl.ds(base, CH)],
                            sidx.at[ncc % 2], si).wait()
                        pltpu.make_async_copy(
                            dst_hbm.at[pl.ds(base, CH)],
                            didx.at[ncc % 2], si).wait()
                    pltpu.async_copy(
                        g_hbm.at[sidx.at[ncc % 2, nj % CH]], bufs.at[b],
                        sems[b])

        plsc.subcore_barrier()
        pltpu.sync_copy(acc_sh.at[pl.ds(sid * TROWS, TROWS)],
                        out_hbm.at[cid, pl.ds(sid * TROWS, TROWS)])

    return deg_kernel, agg_kernel


def _b1_body(x_ref, w_ref, dg_ref, g_ref, dv_ref):
    deg = dg_ref[0] + dg_ref[1] + 1.0
    dinv = lax.rsqrt(deg)
    h = jnp.dot(x_ref[...], w_ref[...], preferred_element_type=jnp.float32)
    g_ref[...] = h * dinv[:, 0:1]
    dv_ref[...] = dinv


_b1 = pl.pallas_call(
    _b1_body,
    grid=(GRID,),
    in_specs=[
        pl.BlockSpec((ROWB, D), lambda i: (i, 0)),
        pl.BlockSpec((D, D), lambda i: (0, 0)),
        pl.BlockSpec((2, ROWB, DEGW), lambda i: (0, i, 0)),
    ],
    out_specs=[
        pl.BlockSpec((ROWB, D), lambda i: (i, 0)),
        pl.BlockSpec((ROWB, DEGW), lambda i: (i, 0)),
    ],
    out_shape=[
        jax.ShapeDtypeStruct((NP, D), jnp.float32),
        jax.ShapeDtypeStruct((NP, DEGW), jnp.float32),
    ],
)


def _mid_body(acc_ref, g_ref, dv_ref, b_ref, w_ref, o_ref):
    dinv = dv_ref[:, 0:1]
    xn = jnp.maximum(
        dinv * (acc_ref[0] + acc_ref[1] + g_ref[...]) + b_ref[...], 0.0)
    o_ref[...] = jnp.dot(
        xn, w_ref[...], preferred_element_type=jnp.float32) * dinv


_mid = pl.pallas_call(
    _mid_body,
    grid=(GRID,),
    in_specs=[
        pl.BlockSpec((2, ROWB, D), lambda i: (0, i, 0)),
        pl.BlockSpec((ROWB, D), lambda i: (i, 0)),
        pl.BlockSpec((ROWB, DEGW), lambda i: (i, 0)),
        pl.BlockSpec((1, D), lambda i: (0, 0)),
        pl.BlockSpec((D, D), lambda i: (0, 0)),
    ],
    out_specs=pl.BlockSpec((ROWB, D), lambda i: (i, 0)),
    out_shape=jax.ShapeDtypeStruct((NP, D), jnp.float32),
)


def _head_body(acc_ref, g_ref, dv_ref, b_ref, lw1_ref, lb1_ref, lw2_ref,
               lb2_ref, o_ref):
    dinv = dv_ref[:, 0:1]
    x4 = dinv * (acc_ref[0] + acc_ref[1] + g_ref[...]) + b_ref[...]
    t = jnp.maximum(
        jnp.dot(x4, lw1_ref[...], preferred_element_type=jnp.float32)
        + lb1_ref[...], 0.0)
    y = jnp.sum(t * lw2_ref[...], axis=1, keepdims=True) + lb2_ref[...]
    o_ref[...] = jax.nn.sigmoid(y)


_head = pl.pallas_call(
    _head_body,
    grid=(GRID,),
    in_specs=[
        pl.BlockSpec((2, ROWB, D), lambda i: (0, i, 0)),
        pl.BlockSpec((ROWB, D), lambda i: (i, 0)),
        pl.BlockSpec((ROWB, DEGW), lambda i: (i, 0)),
        pl.BlockSpec((1, D), lambda i: (0, 0)),
        pl.BlockSpec((D, D // 2), lambda i: (0, 0)),
        pl.BlockSpec((1, D // 2), lambda i: (0, 0)),
        pl.BlockSpec((1, D // 2), lambda i: (0, 0)),
        pl.BlockSpec((1, 1), lambda i: (0, 0)),
    ],
    out_specs=pl.BlockSpec((ROWB, 1), lambda i: (i, 0)),
    out_shape=jax.ShapeDtypeStruct((NP, 1), jnp.float32),
)


def kernel(x, edge_index, batch, W1, b1, W2, b2, W3, b3, lw1, lb1, lw2, lb2):
    del batch
    xp = jnp.pad(x, ((0, NP - N), (0, 0)))
    pad = EP - E
    srcp = jnp.concatenate(
        [edge_index[0], jnp.zeros((pad,), jnp.int32)]).reshape(EPR, 128)
    dstp = jnp.concatenate(
        [edge_index[1], jnp.full((pad,), N, jnp.int32)]).reshape(EPR, 128)

    deg_kernel, agg_kernel = _sc_kernels()
    degp = deg_kernel(dstp)
    g1, dinv = _b1(xp, W1, degp)
    acc1 = agg_kernel(g1, srcp, dstp)
    g2 = _mid(acc1, g1, dinv, b1.reshape(1, D), W2)
    acc2 = agg_kernel(g2, srcp, dstp)
    g3 = _mid(acc2, g2, dinv, b2.reshape(1, D), W3)
    acc3 = agg_kernel(g3, srcp, dstp)
    y = _head(acc3, g3, dinv, b3.reshape(1, D), lw1,
              lb1.reshape(1, -1), lw2.reshape(1, -1), lb2.reshape(1, 1))
    return y[:N]


# speedup vs baseline: 12.3063x; 1.3189x over previous
import functools

import jax
import jax.numpy as jnp
from jax import lax
from jax.experimental import pallas as pl
from jax.experimental.pallas import tpu as pltpu
from jax.experimental.pallas import tpu_sc as plsc

N = 10000
D = 128
FH = D // 2
E = 320000
NP = 10240
EP = 327680
EPR = EP // 128
NBLK = EPR // 16
CH = 8
NCH = NBLK // CH
NBUF = 8
AH = 4
TROWS = NP // 16
DEGW = 16
ROWB = 1024
GRID = NP // ROWB


@functools.lru_cache(maxsize=None)
def _sc_kernels():
    mesh = plsc.VectorSubcoreMesh(
        core_axis_name="c", subcore_axis_name="s", num_cores=2,
        num_subcores=16)

    @functools.partial(
        pl.kernel,
        out_type=jax.ShapeDtypeStruct((2, NP, DEGW), jnp.float32),
        mesh=mesh,
        scratch_types=[
            pltpu.VMEM_SHARED((NP, DEGW), jnp.float32),
            pltpu.VMEM((NBLK // 2, 128), jnp.int32),
            pltpu.VMEM((128, DEGW), jnp.float32),
        ],
        compiler_params=pltpu.CompilerParams(use_tc_tiling_on_sc=False),
    )
    def deg_kernel(dst_hbm, out_hbm, deg_sh, idx_v, val_v):
        cid = lax.axis_index("c")
        sid = lax.axis_index("s")
        wid = sid * 2 + cid
        nb = NBLK // 2

        def _fill(v):
            def body(i, carry):
                val_v[i, :] = jnp.full((DEGW,), v, jnp.float32)
                return carry
            lax.fori_loop(0, 128, body, 0)

        _fill(0.0)
        for k in range(TROWS // 128):
            pltpu.sync_copy(val_v, deg_sh.at[pl.ds(sid * TROWS + k * 128, 128)])
        _fill(1.0)
        pltpu.sync_copy(dst_hbm.at[pl.ds(wid * nb, nb)], idx_v)
        plsc.subcore_barrier()

        def body(j, carry):
            pltpu.sync_copy(val_v, deg_sh.at[idx_v.at[j]], add=True)
            return carry
        lax.fori_loop(0, nb, body, 0)

        plsc.subcore_barrier()
        pltpu.sync_copy(deg_sh.at[pl.ds(sid * TROWS, TROWS)],
                        out_hbm.at[cid, pl.ds(sid * TROWS, TROWS)])

    @functools.partial(
        pl.kernel,
        out_type=jax.ShapeDtypeStruct((2, NP, FH), jnp.float32),
        mesh=mesh,
        scratch_types=(
            [
                pltpu.VMEM_SHARED((NP, FH), jnp.float32),
                pltpu.VMEM((3, CH, 128), jnp.int32),
                pltpu.VMEM((3, CH, 128), jnp.int32),
                pltpu.VMEM((NBUF, 128, FH), jnp.float32),
            ]
            + [pltpu.SemaphoreType.DMA] * NBUF
            + [pltpu.SemaphoreType.DMA] * NBUF
            + [pltpu.SemaphoreType.DMA]
        ),
        compiler_params=pltpu.CompilerParams(use_tc_tiling_on_sc=False),
    )
    def agg_kernel(g_hbm, src_hbm, dst_hbm, out_hbm,
                   acc_sh, sidx, didx, bufs, *sems):
        gsem = sems[:NBUF]
        ssem = sems[NBUF:2 * NBUF]
        si = sems[2 * NBUF]
        cid = lax.axis_index("c")
        sid = lax.axis_index("s")
        base = sid * NBLK
        goff = cid * NP

        def zrow(r, carry):
            def zcol(c_, inner):
                bufs[0, r, pl.ds(c_ * 16, 16)] = jnp.zeros((16,), jnp.float32)
                return inner
            lax.fori_loop(0, FH // 16, zcol, 0)
            return carry
        lax.fori_loop(0, 128, zrow, 0)
        for k in range(TROWS // 128):
            pltpu.sync_copy(bufs.at[0],
                            acc_sh.at[pl.ds(sid * TROWS + k * 128, 128)])

        def add_goff(p):
            def orow(r, carry):
                def ocol(k_, inner):
                    sl = sidx[p, r, pl.ds(k_ * 16, 16)]
                    sidx[p, r, pl.ds(k_ * 16, 16)] = sl + goff
                    return inner
                lax.fori_loop(0, 128 // 16, ocol, 0)
                return carry
            lax.fori_loop(0, CH, orow, 0)

        def fire_idx(c, p):
            pltpu.async_copy(src_hbm.at[pl.ds(base + c * CH, CH)],
                             sidx.at[p], si)
            pltpu.async_copy(dst_hbm.at[pl.ds(base + c * CH, CH)],
                             didx.at[p], si)

        def wait_idx(p):
            pltpu.make_async_copy(
                src_hbm.at[pl.ds(base, CH)], sidx.at[p], si).wait()
            pltpu.make_async_copy(
                dst_hbm.at[pl.ds(base, CH)], didx.at[p], si).wait()
            add_goff(p)

        def fire_gather(p, r, b):
            pltpu.async_copy(g_hbm.at[sidx.at[p, r]], bufs.at[b], gsem[b])

        def wait_gather(p, r, b):
            pltpu.make_async_copy(
                g_hbm.at[sidx.at[p, r]], bufs.at[b], gsem[b]).wait()

        def fire_scatter(p, r, b):
            pltpu.async_copy(bufs.at[b], acc_sh.at[didx.at[p, r]], ssem[b],
                             add=True)

        def wait_scatter(p, r, b):
            pltpu.make_async_copy(
                bufs.at[b], acc_sh.at[didx.at[p, r]], ssem[b]).wait()

        pltpu.sync_copy(src_hbm.at[pl.ds(base, CH)], sidx.at[0])
        pltpu.sync_copy(dst_hbm.at[pl.ds(base, CH)], didx.at[0])
        add_goff(0)
        fire_idx(1, 1)
        plsc.subcore_barrier()
        for b in range(AH):
            fire_gather(0, b, b)

        for j0 in range(CH):
            wait_gather(0, j0, j0)
            fire_scatter(0, j0, j0)
            nj = j0 + AH
            b2 = nj % NBUF
            if nj < CH:
                fire_gather(0, nj, b2)
            else:
                if j0 == AH:
                    wait_idx(1)
                wait_scatter(0, j0 - AH, b2)
                fire_gather(1, nj - CH, b2)

        def chunk_body(c, carry):
            p = lax.rem(c, 3)
            pn = lax.rem(c + 1, 3)
            pv = lax.rem(c + 2, 3)
            for j0 in range(CH):
                wait_gather(p, j0, j0)
                fire_scatter(p, j0, j0)
                if j0 == 0:
                    fire_idx(c + 1, pn)
                b2 = (j0 + AH) % NBUF
                if j0 < AH:
                    wait_scatter(pv, j0 + AH, b2)
                    fire_gather(p, j0 + AH, b2)
                else:
                    if j0 == AH:
                        wait_idx(pn)
                    wait_scatter(p, j0 - AH, b2)
                    fire_gather(pn, j0 - AH, b2)
            return carry
        lax.fori_loop(1, NCH - 1, chunk_body, 0)

        pl_ = (NCH - 1) % 3
        pv_ = (NCH - 2) % 3
        for j0 in range(CH):
            wait_gather(pl_, j0, j0)
            fire_scatter(pl_, j0, j0)
            if j0 < AH:
                b2 = (j0 + AH) % NBUF
                wait_scatter(pv_, j0 + AH, b2)
                fire_gather(pl_, j0 + AH, b2)
        for j0 in range(CH):
            wait_scatter(pl_, j0, j0)

        plsc.subcore_barrier()
        pltpu.sync_copy(acc_sh.at[pl.ds(sid * TROWS, TROWS)],
                        out_hbm.at[cid, pl.ds(sid * TROWS, TROWS)])

    return deg_kernel, agg_kernel


def _b1_body(x_ref, w_ref, dg_ref, g_ref, dv_ref):
    deg = dg_ref[0] + dg_ref[1] + 1.0
    dinv = lax.rsqrt(deg)
    h = jnp.dot(x_ref[...], w_ref[...], preferred_element_type=jnp.float32)
    g_ref[0] = h[:, :FH] * dinv[:, 0:1]
    g_ref[1] = h[:, FH:] * dinv[:, 0:1]
    dv_ref[...] = dinv


_b1 = pl.pallas_call(
    _b1_body,
    grid=(GRID,),
    in_specs=[
        pl.BlockSpec((ROWB, D), lambda i: (i, 0)),
        pl.BlockSpec((D, D), lambda i: (0, 0)),
        pl.BlockSpec((2, ROWB, DEGW), lambda i: (0, i, 0)),
    ],
    out_specs=[
        pl.BlockSpec((2, ROWB, FH), lambda i: (0, i, 0)),
        pl.BlockSpec((ROWB, DEGW), lambda i: (i, 0)),
    ],
    out_shape=[
        jax.ShapeDtypeStruct((2, NP, FH), jnp.float32),
        jax.ShapeDtypeStruct((NP, DEGW), jnp.float32),
    ],
)


def _mid_body(acc_ref, g_ref, dv_ref, b_ref, w_ref, o_ref):
    dinv = dv_ref[:, 0:1]
    x0 = jnp.maximum(dinv * (acc_ref[0] + g_ref[0]) + b_ref[0], 0.0)
    x1 = jnp.maximum(dinv * (acc_ref[1] + g_ref[1]) + b_ref[1], 0.0)
    xn = jnp.concatenate([x0, x1], axis=1)
    h = jnp.dot(xn, w_ref[...], preferred_element_type=jnp.float32)
    o_ref[0] = h[:, :FH] * dinv
    o_ref[1] = h[:, FH:] * dinv


_mid = pl.pallas_call(
    _mid_body,
    grid=(GRID,),
    in_specs=[
        pl.BlockSpec((2, ROWB, FH), lambda i: (0, i, 0)),
        pl.BlockSpec((2, ROWB, FH), lambda i: (0, i, 0)),
        pl.BlockSpec((ROWB, DEGW), lambda i: (i, 0)),
        pl.BlockSpec((2, 1, FH), lambda i: (0, 0, 0)),
        pl.BlockSpec((D, D), lambda i: (0, 0)),
    ],
    out_specs=pl.BlockSpec((2, ROWB, FH), lambda i: (0, i, 0)),
    out_shape=jax.ShapeDtypeStruct((2, NP, FH), jnp.float32),
)


def _head_body(acc_ref, g_ref, dv_ref, b_ref, lw1_ref, lb1_ref, lw2_ref,
               lb2_ref, o_ref):
    dinv = dv_ref[:, 0:1]
    x0 = dinv * (acc_ref[0] + g_ref[0]) + b_ref[0]
    x1 = dinv * (acc_ref[1] + g_ref[1]) + b_ref[1]
    x4 = jnp.concatenate([x0, x1], axis=1)
    t = jnp.maximum(
        jnp.dot(x4, lw1_ref[...], preferred_element_type=jnp.float32)
        + lb1_ref[...], 0.0)
    y = jnp.sum(t * lw2_ref[...], axis=1, keepdims=True) + lb2_ref[...]
    o_ref[...] = jax.nn.sigmoid(y)


_head = pl.pallas_call(
    _head_body,
    grid=(GRID,),
    in_specs=[
        pl.BlockSpec((2, ROWB, FH), lambda i: (0, i, 0)),
        pl.BlockSpec((2, ROWB, FH), lambda i: (0, i, 0)),
        pl.BlockSpec((ROWB, DEGW), lambda i: (i, 0)),
        pl.BlockSpec((2, 1, FH), lambda i: (0, 0, 0)),
        pl.BlockSpec((D, D // 2), lambda i: (0, 0)),
        pl.BlockSpec((1, D // 2), lambda i: (0, 0)),
        pl.BlockSpec((1, D // 2), lambda i: (0, 0)),
        pl.BlockSpec((1, 1), lambda i: (0, 0)),
    ],
    out_specs=pl.BlockSpec((ROWB, 1), lambda i: (i, 0)),
    out_shape=jax.ShapeDtypeStruct((NP, 1), jnp.float32),
)


def kernel(x, edge_index, batch, W1, b1, W2, b2, W3, b3, lw1, lb1, lw2, lb2):
    del batch
    xp = jnp.pad(x, ((0, NP - N), (0, 0)))
    pad = EP - E
    srcp = jnp.concatenate(
        [edge_index[0], jnp.zeros((pad,), jnp.int32)]).reshape(EPR, 128)
    dstp = jnp.concatenate(
        [edge_index[1], jnp.full((pad,), N, jnp.int32)]).reshape(EPR, 128)

    deg_kernel, agg_kernel = _sc_kernels()
    degp = deg_kernel(dstp)
    g1, dinv = _b1(xp, W1, degp)
    acc1 = agg_kernel(g1.reshape(2 * NP, FH), srcp, dstp)
    g2 = _mid(acc1, g1, dinv, b1.reshape(2, 1, FH), W2)
    acc2 = agg_kernel(g2.reshape(2 * NP, FH), srcp, dstp)
    g3 = _mid(acc2, g2, dinv, b2.reshape(2, 1, FH), W3)
    acc3 = agg_kernel(g3.reshape(2 * NP, FH), srcp, dstp)
    y = _head(acc3, g3, dinv, b3.reshape(2, 1, FH), lw1,
              lb1.reshape(1, -1), lw2.reshape(1, -1), lb2.reshape(1, 1))
    return y[:N]


# speedup vs baseline: 22.8703x; 1.8584x over previous
import functools

import jax
import jax.numpy as jnp
from jax import lax
from jax.experimental import pallas as pl
from jax.experimental.pallas import tpu as pltpu
from jax.experimental.pallas import tpu_sc as plsc

N = 10000
D = 128
FH = D // 2
E = 320000
NP = 10240
EP = 327680
EPR = EP // 128
NBLK = EPR // 16
CH = 8
NCH = NBLK // CH
NBUF = 4
AH = 2
SB = NBUF - AH
TROWS = NP // 16
DEGW = 16
ROWB = 1024
GRID = NP // ROWB


@functools.lru_cache(maxsize=None)
def _sc_kernels():
    mesh = plsc.VectorSubcoreMesh(
        core_axis_name="c", subcore_axis_name="s", num_cores=2,
        num_subcores=16)

    @functools.partial(
        pl.kernel,
        out_type=jax.ShapeDtypeStruct((2, NP, DEGW), jnp.float32),
        mesh=mesh,
        scratch_types=[
            pltpu.VMEM_SHARED((NP, DEGW), jnp.float32),
            pltpu.VMEM((NBLK // 2, 128), jnp.int32),
            pltpu.VMEM((128, DEGW), jnp.float32),
        ],
        compiler_params=pltpu.CompilerParams(use_tc_tiling_on_sc=False),
    )
    def deg_kernel(dst_hbm, out_hbm, deg_sh, idx_v, val_v):
        cid = lax.axis_index("c")
        sid = lax.axis_index("s")
        wid = sid * 2 + cid
        nb = NBLK // 2

        def _fill(v):
            def body(i, carry):
                val_v[i, :] = jnp.full((DEGW,), v, jnp.float32)
                return carry
            lax.fori_loop(0, 128, body, 0)

        _fill(0.0)
        for k in range(TROWS // 128):
            pltpu.sync_copy(val_v, deg_sh.at[pl.ds(sid * TROWS + k * 128, 128)])
        _fill(1.0)
        pltpu.sync_copy(dst_hbm.at[pl.ds(wid * nb, nb)], idx_v)
        plsc.subcore_barrier()

        def body(j, carry):
            pltpu.sync_copy(val_v, deg_sh.at[idx_v.at[j]], add=True)
            return carry
        lax.fori_loop(0, nb, body, 0)

        plsc.subcore_barrier()
        pltpu.sync_copy(deg_sh.at[pl.ds(sid * TROWS, TROWS)],
                        out_hbm.at[cid, pl.ds(sid * TROWS, TROWS)])

    @functools.partial(
        pl.kernel,
        out_type=jax.ShapeDtypeStruct((2, NP, FH), jnp.float32),
        mesh=mesh,
        scratch_types=(
            [
                pltpu.VMEM_SHARED((NP, FH), jnp.float32),
                pltpu.VMEM_SHARED((NP, FH), jnp.float32),
                pltpu.VMEM((3, CH, 128), jnp.int32),
                pltpu.VMEM((3, CH, 128), jnp.int32),
                pltpu.VMEM((NBUF, 128, FH), jnp.float32),
            ]
            + [pltpu.SemaphoreType.DMA] * NBUF
            + [pltpu.SemaphoreType.DMA] * NBUF
            + [pltpu.SemaphoreType.DMA]
        ),
        compiler_params=pltpu.CompilerParams(use_tc_tiling_on_sc=False),
    )
    def agg_kernel(g_hbm, src_hbm, dst_hbm, out_hbm,
                   acc_sh, gtab, sidx, didx, bufs, *sems):
        gsem = sems[:NBUF]
        ssem = sems[NBUF:2 * NBUF]
        si = sems[2 * NBUF]
        cid = lax.axis_index("c")
        sid = lax.axis_index("s")
        base = sid * NBLK
        goff = cid * NP

        def zrow(r, carry):
            def zcol(c_, inner):
                bufs[0, r, pl.ds(c_ * 16, 16)] = jnp.zeros((16,), jnp.float32)
                return inner
            lax.fori_loop(0, FH // 16, zcol, 0)
            return carry
        lax.fori_loop(0, 128, zrow, 0)
        for k in range(TROWS // 128):
            pltpu.sync_copy(bufs.at[0],
                            acc_sh.at[pl.ds(sid * TROWS + k * 128, 128)])
        pltpu.sync_copy(g_hbm.at[pl.ds(goff + sid * TROWS, TROWS)],
                        gtab.at[pl.ds(sid * TROWS, TROWS)])

        def fire_idx(c, p):
            pltpu.async_copy(src_hbm.at[pl.ds(base + c * CH, CH)],
                             sidx.at[p], si)
            pltpu.async_copy(dst_hbm.at[pl.ds(base + c * CH, CH)],
                             didx.at[p], si)

        def wait_idx(p):
            pltpu.make_async_copy(
                src_hbm.at[pl.ds(base, CH)], sidx.at[p], si).wait()
            pltpu.make_async_copy(
                dst_hbm.at[pl.ds(base, CH)], didx.at[p], si).wait()

        def fire_gather(p, r, b):
            pltpu.async_copy(gtab.at[sidx.at[p, r]], bufs.at[b], gsem[b])

        def wait_gather(p, r, b):
            pltpu.make_async_copy(
                gtab.at[sidx.at[p, r]], bufs.at[b], gsem[b]).wait()

        def fire_scatter(p, r, b):
            pltpu.async_copy(bufs.at[b], acc_sh.at[didx.at[p, r]], ssem[b],
                             add=True)

        def wait_scatter(p, r, b):
            pltpu.make_async_copy(
                bufs.at[b], acc_sh.at[didx.at[p, r]], ssem[b]).wait()

        pltpu.sync_copy(src_hbm.at[pl.ds(base, CH)], sidx.at[0])
        pltpu.sync_copy(dst_hbm.at[pl.ds(base, CH)], didx.at[0])
        fire_idx(1, 1)
        plsc.subcore_barrier()
        for b in range(AH):
            fire_gather(0, b, b)

        for j0 in range(CH):
            b = j0 % NBUF
            wait_gather(0, j0, b)
            fire_scatter(0, j0, b)
            b2 = (j0 + AH) % NBUF
            if j0 >= SB:
                wait_scatter(0, j0 - SB, b2)
            if j0 < CH - AH:
                fire_gather(0, j0 + AH, b2)
            else:
                if j0 == CH - AH:
                    wait_idx(1)
                fire_gather(1, j0 - (CH - AH), b2)

        def chunk_body(c, carry):
            p = lax.rem(c, 3)
            pn = lax.rem(c + 1, 3)
            pv = lax.rem(c + 2, 3)
            for j0 in range(CH):
                b = j0 % NBUF
                wait_gather(p, j0, b)
                fire_scatter(p, j0, b)
                if j0 == 0:
                    fire_idx(c + 1, pn)
                b2 = (j0 + AH) % NBUF
                if j0 < SB:
                    wait_scatter(pv, j0 + CH - SB, b2)
                else:
                    wait_scatter(p, j0 - SB, b2)
                if j0 < CH - AH:
                    fire_gather(p, j0 + AH, b2)
                else:
                    if j0 == CH - AH:
                        wait_idx(pn)
                    fire_gather(pn, j0 - (CH - AH), b2)
            return carry
        lax.fori_loop(1, NCH - 1, chunk_body, 0)

        pl_ = (NCH - 1) % 3
        pv_ = (NCH - 2) % 3
        for j0 in range(CH):
            b = j0 % NBUF
            wait_gather(pl_, j0, b)
            fire_scatter(pl_, j0, b)
            if j0 < CH - AH:
                b2 = (j0 + AH) % NBUF
                if j0 < SB:
                    wait_scatter(pv_, j0 + CH - SB, b2)
                else:
                    wait_scatter(pl_, j0 - SB, b2)
                fire_gather(pl_, j0 + AH, b2)
        for j0 in range(CH - NBUF, CH):
            wait_scatter(pl_, j0, j0 % NBUF)

        plsc.subcore_barrier()
        pltpu.sync_copy(acc_sh.at[pl.ds(sid * TROWS, TROWS)],
                        out_hbm.at[cid, pl.ds(sid * TROWS, TROWS)])

    return deg_kernel, agg_kernel


def _b1_body(x_ref, w_ref, dg_ref, g_ref, dv_ref):
    deg = dg_ref[0] + dg_ref[1] + 1.0
    dinv = lax.rsqrt(deg)
    h = jnp.dot(x_ref[...], w_ref[...], preferred_element_type=jnp.float32)
    g_ref[0] = h[:, :FH] * dinv[:, 0:1]
    g_ref[1] = h[:, FH:] * dinv[:, 0:1]
    dv_ref[...] = dinv


_b1 = pl.pallas_call(
    _b1_body,
    grid=(GRID,),
    in_specs=[
        pl.BlockSpec((ROWB, D), lambda i: (i, 0)),
        pl.BlockSpec((D, D), lambda i: (0, 0)),
        pl.BlockSpec((2, ROWB, DEGW), lambda i: (0, i, 0)),
    ],
    out_specs=[
        pl.BlockSpec((2, ROWB, FH), lambda i: (0, i, 0)),
        pl.BlockSpec((ROWB, DEGW), lambda i: (i, 0)),
    ],
    out_shape=[
        jax.ShapeDtypeStruct((2, NP, FH), jnp.float32),
        jax.ShapeDtypeStruct((NP, DEGW), jnp.float32),
    ],
)


def _mid_body(acc_ref, g_ref, dv_ref, b_ref, w_ref, o_ref):
    dinv = dv_ref[:, 0:1]
    x0 = jnp.maximum(dinv * (acc_ref[0] + g_ref[0]) + b_ref[0], 0.0)
    x1 = jnp.maximum(dinv * (acc_ref[1] + g_ref[1]) + b_ref[1], 0.0)
    xn = jnp.concatenate([x0, x1], axis=1)
    h = jnp.dot(xn, w_ref[...], preferred_element_type=jnp.float32)
    o_ref[0] = h[:, :FH] * dinv
    o_ref[1] = h[:, FH:] * dinv


_mid = pl.pallas_call(
    _mid_body,
    grid=(GRID,),
    in_specs=[
        pl.BlockSpec((2, ROWB, FH), lambda i: (0, i, 0)),
        pl.BlockSpec((2, ROWB, FH), lambda i: (0, i, 0)),
        pl.BlockSpec((ROWB, DEGW), lambda i: (i, 0)),
        pl.BlockSpec((2, 1, FH), lambda i: (0, 0, 0)),
        pl.BlockSpec((D, D), lambda i: (0, 0)),
    ],
    out_specs=pl.BlockSpec((2, ROWB, FH), lambda i: (0, i, 0)),
    out_shape=jax.ShapeDtypeStruct((2, NP, FH), jnp.float32),
)


def _head_body(acc_ref, g_ref, dv_ref, b_ref, lw1_ref, lb1_ref, lw2_ref,
               lb2_ref, o_ref):
    dinv = dv_ref[:, 0:1]
    x0 = dinv * (acc_ref[0] + g_ref[0]) + b_ref[0]
    x1 = dinv * (acc_ref[1] + g_ref[1]) + b_ref[1]
    x4 = jnp.concatenate([x0, x1], axis=1)
    t = jnp.maximum(
        jnp.dot(x4, lw1_ref[...], preferred_element_type=jnp.float32)
        + lb1_ref[...], 0.0)
    y = jnp.sum(t * lw2_ref[...], axis=1, keepdims=True) + lb2_ref[...]
    o_ref[...] = jax.nn.sigmoid(y)


_head = pl.pallas_call(
    _head_body,
    grid=(GRID,),
    in_specs=[
        pl.BlockSpec((2, ROWB, FH), lambda i: (0, i, 0)),
        pl.BlockSpec((2, ROWB, FH), lambda i: (0, i, 0)),
        pl.BlockSpec((ROWB, DEGW), lambda i: (i, 0)),
        pl.BlockSpec((2, 1, FH), lambda i: (0, 0, 0)),
        pl.BlockSpec((D, D // 2), lambda i: (0, 0)),
        pl.BlockSpec((1, D // 2), lambda i: (0, 0)),
        pl.BlockSpec((1, D // 2), lambda i: (0, 0)),
        pl.BlockSpec((1, 1), lambda i: (0, 0)),
    ],
    out_specs=pl.BlockSpec((ROWB, 1), lambda i: (i, 0)),
    out_shape=jax.ShapeDtypeStruct((NP, 1), jnp.float32),
)


def kernel(x, edge_index, batch, W1, b1, W2, b2, W3, b3, lw1, lb1, lw2, lb2):
    del batch
    xp = jnp.pad(x, ((0, NP - N), (0, 0)))
    pad = EP - E
    srcp = jnp.concatenate(
        [edge_index[0], jnp.zeros((pad,), jnp.int32)]).reshape(EPR, 128)
    dstp = jnp.concatenate(
        [edge_index[1], jnp.full((pad,), N, jnp.int32)]).reshape(EPR, 128)

    deg_kernel, agg_kernel = _sc_kernels()
    degp = deg_kernel(dstp)
    g1, dinv = _b1(xp, W1, degp)
    acc1 = agg_kernel(g1.reshape(2 * NP, FH), srcp, dstp)
    g2 = _mid(acc1, g1, dinv, b1.reshape(2, 1, FH), W2)
    acc2 = agg_kernel(g2.reshape(2 * NP, FH), srcp, dstp)
    g3 = _mid(acc2, g2, dinv, b2.reshape(2, 1, FH), W3)
    acc3 = agg_kernel(g3.reshape(2 * NP, FH), srcp, dstp)
    y = _head(acc3, g3, dinv, b3.reshape(2, 1, FH), lw1,
              lb1.reshape(1, -1), lw2.reshape(1, -1), lb2.reshape(1, 1))
    return y[:N]


# speedup vs baseline: 22.9005x; 1.0013x over previous
import functools

import jax
import jax.numpy as jnp
from jax import lax
from jax.experimental import pallas as pl
from jax.experimental.pallas import tpu as pltpu
from jax.experimental.pallas import tpu_sc as plsc

N = 10000
D = 128
FH = D // 2
E = 320000
NP = 10240
EP = 327680
EPR = EP // 128
NBLK = EPR // 16
CH = 8
NCH = NBLK // CH
NBUF = 4
AH = 3
SB = NBUF - AH
TROWS = NP // 16
DEGW = 16
ROWB = 1024
GRID = NP // ROWB


@functools.lru_cache(maxsize=None)
def _sc_kernels():
    mesh = plsc.VectorSubcoreMesh(
        core_axis_name="c", subcore_axis_name="s", num_cores=2,
        num_subcores=16)

    @functools.partial(
        pl.kernel,
        out_type=jax.ShapeDtypeStruct((2, NP, DEGW), jnp.float32),
        mesh=mesh,
        scratch_types=[
            pltpu.VMEM_SHARED((NP, DEGW), jnp.float32),
            pltpu.VMEM((NBLK // 2, 128), jnp.int32),
            pltpu.VMEM((128, DEGW), jnp.float32),
        ],
        compiler_params=pltpu.CompilerParams(use_tc_tiling_on_sc=False),
    )
    def deg_kernel(dst_hbm, out_hbm, deg_sh, idx_v, val_v):
        cid = lax.axis_index("c")
        sid = lax.axis_index("s")
        wid = sid * 2 + cid
        nb = NBLK // 2

        def _fill(v):
            def body(i, carry):
                val_v[i, :] = jnp.full((DEGW,), v, jnp.float32)
                return carry
            lax.fori_loop(0, 128, body, 0)

        _fill(0.0)
        for k in range(TROWS // 128):
            pltpu.sync_copy(val_v, deg_sh.at[pl.ds(sid * TROWS + k * 128, 128)])
        _fill(1.0)
        pltpu.sync_copy(dst_hbm.at[pl.ds(wid * nb, nb)], idx_v)
        plsc.subcore_barrier()

        def body(j, carry):
            pltpu.sync_copy(val_v, deg_sh.at[idx_v.at[j]], add=True)
            return carry
        lax.fori_loop(0, nb, body, 0)

        plsc.subcore_barrier()
        pltpu.sync_copy(deg_sh.at[pl.ds(sid * TROWS, TROWS)],
                        out_hbm.at[cid, pl.ds(sid * TROWS, TROWS)])

    @functools.partial(
        pl.kernel,
        out_type=jax.ShapeDtypeStruct((2, NP, FH), jnp.float32),
        mesh=mesh,
        scratch_types=(
            [
                pltpu.VMEM_SHARED((NP, FH), jnp.float32),
                pltpu.VMEM_SHARED((NP, FH), jnp.float32),
                pltpu.VMEM((3, CH, 128), jnp.int32),
                pltpu.VMEM((3, CH, 128), jnp.int32),
                pltpu.VMEM((NBUF, 128, FH), jnp.float32),
            ]
            + [pltpu.SemaphoreType.DMA] * NBUF
            + [pltpu.SemaphoreType.DMA] * NBUF
            + [pltpu.SemaphoreType.DMA]
        ),
        compiler_params=pltpu.CompilerParams(use_tc_tiling_on_sc=False),
    )
    def agg_kernel(g_hbm, src_hbm, dst_hbm, out_hbm,
                   acc_sh, gtab, sidx, didx, bufs, *sems):
        gsem = sems[:NBUF]
        ssem = sems[NBUF:2 * NBUF]
        si = sems[2 * NBUF]
        cid = lax.axis_index("c")
        sid = lax.axis_index("s")
        base = sid * NBLK
        goff = cid * NP

        def zrow(r, carry):
            def zcol(c_, inner):
                bufs[0, r, pl.ds(c_ * 16, 16)] = jnp.zeros((16,), jnp.float32)
                return inner
            lax.fori_loop(0, FH // 16, zcol, 0)
            return carry
        lax.fori_loop(0, 128, zrow, 0)
        for k in range(TROWS // 128):
            pltpu.sync_copy(bufs.at[0],
                            acc_sh.at[pl.ds(sid * TROWS + k * 128, 128)])
        pltpu.sync_copy(g_hbm.at[pl.ds(goff + sid * TROWS, TROWS)],
                        gtab.at[pl.ds(sid * TROWS, TROWS)])

        def fire_idx(c, p):
            pltpu.async_copy(src_hbm.at[pl.ds(base + c * CH, CH)],
                             sidx.at[p], si)
            pltpu.async_copy(dst_hbm.at[pl.ds(base + c * CH, CH)],
                             didx.at[p], si)

        def wait_idx(p):
            pltpu.make_async_copy(
                src_hbm.at[pl.ds(base, CH)], sidx.at[p], si).wait()
            pltpu.make_async_copy(
                dst_hbm.at[pl.ds(base, CH)], didx.at[p], si).wait()

        def fire_gather(p, r, b):
            pltpu.async_copy(gtab.at[sidx.at[p, r]], bufs.at[b], gsem[b])

        def wait_gather(p, r, b):
            pltpu.make_async_copy(
                gtab.at[sidx.at[p, r]], bufs.at[b], gsem[b]).wait()

        def fire_scatter(p, r, b):
            pltpu.async_copy(bufs.at[b], acc_sh.at[didx.at[p, r]], ssem[b],
                             add=True)

        def wait_scatter(p, r, b):
            pltpu.make_async_copy(
                bufs.at[b], acc_sh.at[didx.at[p, r]], ssem[b]).wait()

        pltpu.sync_copy(src_hbm.at[pl.ds(base, CH)], sidx.at[0])
        pltpu.sync_copy(dst_hbm.at[pl.ds(base, CH)], didx.at[0])
        fire_idx(1, 1)
        plsc.subcore_barrier()
        for b in range(AH):
            fire_gather(0, b, b)

        for j0 in range(CH):
            b = j0 % NBUF
            wait_gather(0, j0, b)
            fire_scatter(0, j0, b)
            b2 = (j0 + AH) % NBUF
            if j0 >= SB:
                wait_scatter(0, j0 - SB, b2)
            if j0 < CH - AH:
                fire_gather(0, j0 + AH, b2)
            else:
                if j0 == CH - AH:
                    wait_idx(1)
                fire_gather(1, j0 - (CH - AH), b2)

        def chunk_body(c, carry):
            p = lax.rem(c, 3)
            pn = lax.rem(c + 1, 3)
            pv = lax.rem(c + 2, 3)
            for j0 in range(CH):
                b = j0 % NBUF
                wait_gather(p, j0, b)
                fire_scatter(p, j0, b)
                if j0 == 0:
                    fire_idx(c + 1, pn)
                b2 = (j0 + AH) % NBUF
                if j0 < SB:
                    wait_scatter(pv, j0 + CH - SB, b2)
                else:
                    wait_scatter(p, j0 - SB, b2)
                if j0 < CH - AH:
                    fire_gather(p, j0 + AH, b2)
                else:
                    if j0 == CH - AH:
                        wait_idx(pn)
                    fire_gather(pn, j0 - (CH - AH), b2)
            return carry
        lax.fori_loop(1, NCH - 1, chunk_body, 0)

        pl_ = (NCH - 1) % 3
        pv_ = (NCH - 2) % 3
        for j0 in range(CH):
            b = j0 % NBUF
            wait_gather(pl_, j0, b)
            fire_scatter(pl_, j0, b)
            if j0 < CH - AH:
                b2 = (j0 + AH) % NBUF
                if j0 < SB:
                    wait_scatter(pv_, j0 + CH - SB, b2)
                else:
                    wait_scatter(pl_, j0 - SB, b2)
                fire_gather(pl_, j0 + AH, b2)
        for j0 in range(CH - NBUF, CH):
            wait_scatter(pl_, j0, j0 % NBUF)

        plsc.subcore_barrier()
        pltpu.sync_copy(acc_sh.at[pl.ds(sid * TROWS, TROWS)],
                        out_hbm.at[cid, pl.ds(sid * TROWS, TROWS)])

    return deg_kernel, agg_kernel


def _b1_body(x_ref, w_ref, dg_ref, g_ref, dv_ref):
    deg = dg_ref[0] + dg_ref[1] + 1.0
    dinv = lax.rsqrt(deg)
    h = jnp.dot(x_ref[...], w_ref[...], preferred_element_type=jnp.float32)
    g_ref[0] = h[:, :FH] * dinv[:, 0:1]
    g_ref[1] = h[:, FH:] * dinv[:, 0:1]
    dv_ref[...] = dinv


_b1 = pl.pallas_call(
    _b1_body,
    grid=(GRID,),
    in_specs=[
        pl.BlockSpec((ROWB, D), lambda i: (i, 0)),
        pl.BlockSpec((D, D), lambda i: (0, 0)),
        pl.BlockSpec((2, ROWB, DEGW), lambda i: (0, i, 0)),
    ],
    out_specs=[
        pl.BlockSpec((2, ROWB, FH), lambda i: (0, i, 0)),
        pl.BlockSpec((ROWB, DEGW), lambda i: (i, 0)),
    ],
    out_shape=[
        jax.ShapeDtypeStruct((2, NP, FH), jnp.float32),
        jax.ShapeDtypeStruct((NP, DEGW), jnp.float32),
    ],
)


def _mid_body(acc_ref, g_ref, dv_ref, b_ref, w_ref, o_ref):
    dinv = dv_ref[:, 0:1]
    x0 = jnp.maximum(dinv * (acc_ref[0] + g_ref[0]) + b_ref[0], 0.0)
    x1 = jnp.maximum(dinv * (acc_ref[1] + g_ref[1]) + b_ref[1], 0.0)
    xn = jnp.concatenate([x0, x1], axis=1)
    h = jnp.dot(xn, w_ref[...], preferred_element_type=jnp.float32)
    o_ref[0] = h[:, :FH] * dinv
    o_ref[1] = h[:, FH:] * dinv


_mid = pl.pallas_call(
    _mid_body,
    grid=(GRID,),
    in_specs=[
        pl.BlockSpec((2, ROWB, FH), lambda i: (0, i, 0)),
        pl.BlockSpec((2, ROWB, FH), lambda i: (0, i, 0)),
        pl.BlockSpec((ROWB, DEGW), lambda i: (i, 0)),
        pl.BlockSpec((2, 1, FH), lambda i: (0, 0, 0)),
        pl.BlockSpec((D, D), lambda i: (0, 0)),
    ],
    out_specs=pl.BlockSpec((2, ROWB, FH), lambda i: (0, i, 0)),
    out_shape=jax.ShapeDtypeStruct((2, NP, FH), jnp.float32),
)


def _head_body(acc_ref, g_ref, dv_ref, b_ref, lw1_ref, lb1_ref, lw2_ref,
               lb2_ref, o_ref):
    dinv = dv_ref[:, 0:1]
    x0 = dinv * (acc_ref[0] + g_ref[0]) + b_ref[0]
    x1 = dinv * (acc_ref[1] + g_ref[1]) + b_ref[1]
    x4 = jnp.concatenate([x0, x1], axis=1)
    t = jnp.maximum(
        jnp.dot(x4, lw1_ref[...], preferred_element_type=jnp.float32)
        + lb1_ref[...], 0.0)
    y = jnp.sum(t * lw2_ref[...], axis=1, keepdims=True) + lb2_ref[...]
    o_ref[...] = jax.nn.sigmoid(y)


_head = pl.pallas_call(
    _head_body,
    grid=(GRID,),
    in_specs=[
        pl.BlockSpec((2, ROWB, FH), lambda i: (0, i, 0)),
        pl.BlockSpec((2, ROWB, FH), lambda i: (0, i, 0)),
        pl.BlockSpec((ROWB, DEGW), lambda i: (i, 0)),
        pl.BlockSpec((2, 1, FH), lambda i: (0, 0, 0)),
        pl.BlockSpec((D, D // 2), lambda i: (0, 0)),
        pl.BlockSpec((1, D // 2), lambda i: (0, 0)),
        pl.BlockSpec((1, D // 2), lambda i: (0, 0)),
        pl.BlockSpec((1, 1), lambda i: (0, 0)),
    ],
    out_specs=pl.BlockSpec((ROWB, 1), lambda i: (i, 0)),
    out_shape=jax.ShapeDtypeStruct((NP, 1), jnp.float32),
)


def kernel(x, edge_index, batch, W1, b1, W2, b2, W3, b3, lw1, lb1, lw2, lb2):
    del batch
    xp = jnp.pad(x, ((0, NP - N), (0, 0)))
    pad = EP - E
    srcp = jnp.concatenate(
        [edge_index[0], jnp.zeros((pad,), jnp.int32)]).reshape(EPR, 128)
    dstp = jnp.concatenate(
        [edge_index[1], jnp.full((pad,), N, jnp.int32)]).reshape(EPR, 128)

    deg_kernel, agg_kernel = _sc_kernels()
    degp = deg_kernel(dstp)
    g1, dinv = _b1(xp, W1, degp)
    acc1 = agg_kernel(g1.reshape(2 * NP, FH), srcp, dstp)
    g2 = _mid(acc1, g1, dinv, b1.reshape(2, 1, FH), W2)
    acc2 = agg_kernel(g2.reshape(2 * NP, FH), srcp, dstp)
    g3 = _mid(acc2, g2, dinv, b2.reshape(2, 1, FH), W3)
    acc3 = agg_kernel(g3.reshape(2 * NP, FH), srcp, dstp)
    y = _head(acc3, g3, dinv, b3.reshape(2, 1, FH), lw1,
              lb1.reshape(1, -1), lw2.reshape(1, -1), lb2.reshape(1, 1))
    return y[:N]
